# baseline (speedup 1.0000x reference)
import numpy as np
import jax
import jax.numpy as jnp
from jax import lax
from jax.experimental import pallas as pl
from jax.experimental.pallas import tpu as pltpu

_K = 512
_BORDER = 8
_R_G = 2
_GAUSS = np.exp(-2.0 * (np.arange(-_R_G, _R_G + 1) ** 2)).astype(np.float32)
_ONE_BITS = 0x3F800000
_HALF_BITS = 0x3F000000
_IMGS = 4


def _nms_survivors(x, h, w):
    row = lax.broadcasted_iota(jnp.int32, (h, w), 0)
    col = lax.broadcasted_iota(jnp.int32, (h, w), 1)
    span = jnp.uint32(h - 2 * _BORDER)
    inb = ((row - _BORDER).astype(jnp.uint32) < span) & \
          ((col - _BORDER).astype(jnp.uint32) < span)
    xt = jnp.where(inb, x, 0.0)

    p1 = jnp.maximum(xt, pltpu.roll(xt, w - 1, 1))
    p3 = jnp.maximum(p1, pltpu.roll(p1, w - 2, 1))
    m1 = jnp.maximum(pltpu.roll(p3, 2, 1),
                     pltpu.roll(xt, w - 2, 1))
    q1 = jnp.maximum(m1, pltpu.roll(m1, h - 1, 0))
    q3 = jnp.maximum(q1, pltpu.roll(q1, h - 2, 0))
    mx = jnp.maximum(pltpu.roll(q3, 2, 0),
                     pltpu.roll(m1, h - 2, 0))
    return jnp.where(xt >= mx, xt, 0.0)


def _topk_mask(yi, cnt_ge_star, cnt_gt, t_star, h, w):
    ties = yi == t_star
    row = lax.broadcasted_iota(jnp.int32, (h, w), 0)
    col = lax.broadcasted_iota(jnp.int32, (h, w), 1)
    flat = row * w + col

    def _tie_search():
        need = _K - cnt_gt

        def _bis2(_, lo_hi):
            lo, hi = lo_hi
            mid = (lo + hi) // 2
            take = jnp.sum((ties & (flat <= mid)).astype(jnp.int32)) >= need
            return jnp.where(take, lo, mid + 1), jnp.where(take, mid, hi)

        return lax.fori_loop(0, 18, _bis2,
                             (jnp.int32(0), jnp.int32(h * w - 1)))[1]

    m_star = lax.cond(cnt_ge_star == _K,
                      lambda: jnp.int32(h * w - 1), _tie_search)
    return (yi > t_star) | (ties & (flat <= m_star))


def _gauss5(z, h, w):
    t1 = z * _GAUSS[_R_G]
    for d in range(1, _R_G + 1):
        t1 = t1 + _GAUSS[_R_G + d] * (pltpu.roll(z, d, 1) +
                                      pltpu.roll(z, w - d, 1))
    o = t1 * _GAUSS[_R_G]
    for d in range(1, _R_G + 1):
        o = o + _GAUSS[_R_G + d] * (pltpu.roll(t1, d, 0) +
                                    pltpu.roll(t1, h - d, 0))
    return jnp.clip(o, 0.0, 1.0)


def _body(x_ref, out_ref, tmask_ref, topkv_ref):
    h, w = x_ref.shape[1], x_ref.shape[2]

    yis, red = [], []
    for i in range(_IMGS):
        y = _nms_survivors(x_ref[i], h, w)
        topkv_ref[i] = y
        yi = lax.bitcast_convert_type(y, jnp.int32)
        yis.append(yi)

        q = h // 4
        s0, s1, s2, s3 = (yi[0:q], yi[q:2 * q], yi[2 * q:3 * q],
                          yi[3 * q:4 * q])
        hi01 = jnp.maximum(s0, s1)
        lo01 = jnp.minimum(s0, s1)
        hi23 = jnp.maximum(s2, s3)
        lo23 = jnp.minimum(s2, s3)
        r1 = jnp.maximum(hi01, hi23)
        mid_hi = jnp.minimum(hi01, hi23)
        w01 = hi01 >= hi23
        lo_w = jnp.where(w01, lo01, lo23)
        lo_l = jnp.where(w01, lo23, lo01)
        r2 = jnp.maximum(mid_hi, lo_w)
        r3 = jnp.maximum(jnp.minimum(lo_w, mid_hi), lo_l)
        red.append((r1, r2, jnp.max(r3)))

    def _cnt_ge(i, t):
        r1, r2, m3_cap = red[i]
        return lax.cond(
            t > m3_cap,
            lambda: (jnp.sum((r1 >= t).astype(jnp.int32)) +
                     jnp.sum((r2 >= t).astype(jnp.int32))),
            lambda: jnp.sum((yis[i] >= t).astype(jnp.int32)))

    def _step(i, carry, probe):
        lo, hi, clo, chi = carry
        c = _cnt_ge(i, probe)
        take = c >= _K
        return (jnp.where(take, probe, lo), jnp.where(take, hi, probe),
                jnp.where(take, c, clo), jnp.where(take, chi, c))

    def _bis(_, carry):
        return tuple(_step(i, carry[i], (carry[i][0] + carry[i][1]) // 2)
                     for i in range(_IMGS))

    init0 = (jnp.int32(0), jnp.int32(_ONE_BITS), jnp.int32(h * w),
             jnp.int32(0))
    seeded = tuple(_step(i, init0, jnp.int32(_HALF_BITS))
                   for i in range(_IMGS))
    res = lax.fori_loop(0, 23, _bis, seeded)

    def _refine(i, carry):
        return lax.cond(
            carry[1] - carry[0] > 1,
            lambda: lax.fori_loop(
                0, 7, lambda _, cr: _step(i, cr, (cr[0] + cr[1]) // 2),
                carry),
            lambda: carry)

    res = tuple(_refine(i, res[i]) for i in range(_IMGS))

    for i in range(_IMGS):
        t_star, _, cnt_ge_star, cnt_gt = res[i]
        tmask = _topk_mask(yis[i], cnt_ge_star, cnt_gt, t_star, h, w)
        tmask_ref[i] = tmask.astype(jnp.int8)
        y = lax.bitcast_convert_type(yis[i], jnp.float32)
        out_ref[i] = _gauss5(jnp.where(tmask, y, 0.0), h, w)


def kernel(im1w_score):
    b, h, w, c = im1w_score.shape
    x = im1w_score.reshape(b, h, w)
    spec = pl.BlockSpec((_IMGS, h, w), lambda i: (i, 0, 0))
    out, tmask, topkv = pl.pallas_call(
        _body,
        grid=(b // _IMGS,),
        in_specs=[spec],
        out_specs=[spec, spec, spec],
        out_shape=[
            jax.ShapeDtypeStruct((b, h, w), jnp.float32),
            jax.ShapeDtypeStruct((b, h, w), jnp.int8),
            jax.ShapeDtypeStruct((b, h, w), jnp.float32),
        ],
        compiler_params=pltpu.CompilerParams(
            dimension_semantics=("arbitrary",)),
    )(x)
    return (out.reshape(b, h, w, c),
            tmask.reshape(b, h, w, c).astype(jnp.bool_),
            topkv.reshape(b, h, w, c))


# speedup vs baseline: 1.2998x; 1.2998x over previous
import numpy as np
import jax
import jax.numpy as jnp
from jax import lax
from jax.experimental import pallas as pl
from jax.experimental.pallas import tpu as pltpu

_K = 512
_BORDER = 8
_R_G = 2
_GAUSS = np.exp(-2.0 * (np.arange(-_R_G, _R_G + 1) ** 2)).astype(np.float32)
_ONE_BITS = 0x3F800000
_HALF_BITS = 0x3F000000
_IMGS = 4


def _nms_survivors(x, h, w):
    row = lax.broadcasted_iota(jnp.int32, (h, w), 0)
    col = lax.broadcasted_iota(jnp.int32, (h, w), 1)
    span = jnp.uint32(h - 2 * _BORDER)
    inb = ((row - _BORDER).astype(jnp.uint32) < span) & \
          ((col - _BORDER).astype(jnp.uint32) < span)
    xt = jnp.where(inb, x, 0.0)

    p1 = jnp.maximum(xt, pltpu.roll(xt, w - 1, 1))
    p3 = jnp.maximum(p1, pltpu.roll(p1, w - 2, 1))
    m1 = jnp.maximum(pltpu.roll(p3, 2, 1),
                     pltpu.roll(xt, w - 2, 1))
    q1 = jnp.maximum(m1, pltpu.roll(m1, h - 1, 0))
    q3 = jnp.maximum(q1, pltpu.roll(q1, h - 2, 0))
    mx = jnp.maximum(pltpu.roll(q3, 2, 0),
                     pltpu.roll(m1, h - 2, 0))
    return jnp.where(xt >= mx, xt, 0.0)


def _topk_mask(yi, cnt_ge_star, cnt_gt, t_star, h, w):
    ties = yi == t_star
    row = lax.broadcasted_iota(jnp.int32, (h, w), 0)
    col = lax.broadcasted_iota(jnp.int32, (h, w), 1)
    flat = row * w + col

    def _tie_search():
        need = _K - cnt_gt

        def _bis2(_, lo_hi):
            lo, hi = lo_hi
            mid = (lo + hi) // 2
            take = jnp.sum((ties & (flat <= mid)).astype(jnp.int32)) >= need
            return jnp.where(take, lo, mid + 1), jnp.where(take, mid, hi)

        return lax.fori_loop(0, 18, _bis2,
                             (jnp.int32(0), jnp.int32(h * w - 1)))[1]

    m_star = lax.cond(cnt_ge_star == _K,
                      lambda: jnp.int32(h * w - 1), _tie_search)
    return (yi > t_star) | (ties & (flat <= m_star))


def _gauss5(z, h, w):
    t1 = z * _GAUSS[_R_G]
    for d in range(1, _R_G + 1):
        t1 = t1 + _GAUSS[_R_G + d] * (pltpu.roll(z, d, 1) +
                                      pltpu.roll(z, w - d, 1))
    o = t1 * _GAUSS[_R_G]
    for d in range(1, _R_G + 1):
        o = o + _GAUSS[_R_G + d] * (pltpu.roll(t1, d, 0) +
                                    pltpu.roll(t1, h - d, 0))
    return jnp.clip(o, 0.0, 1.0)


def _body(x_ref, out_ref, tmask_ref, topkv_ref):
    h, w = x_ref.shape[1], x_ref.shape[2]

    yis, red = [], []
    for i in range(_IMGS):
        y = _nms_survivors(x_ref[i], h, w)
        topkv_ref[i] = y
        yi = lax.bitcast_convert_type(y, jnp.int32)
        yis.append(yi)

        q = h // 4
        s0, s1, s2, s3 = (yi[0:q], yi[q:2 * q], yi[2 * q:3 * q],
                          yi[3 * q:4 * q])
        hi01 = jnp.maximum(s0, s1)
        lo01 = jnp.minimum(s0, s1)
        hi23 = jnp.maximum(s2, s3)
        lo23 = jnp.minimum(s2, s3)
        r1 = jnp.maximum(hi01, hi23)
        mid_hi = jnp.minimum(hi01, hi23)
        lo_w = jnp.where(hi01 >= hi23, lo01, lo23)
        r2 = jnp.maximum(mid_hi, lo_w)
        red.append((r1, r2))

    def _cnt_red(i, t):
        r1, r2 = red[i]
        return (jnp.sum((r1 >= t).astype(jnp.int32)) +
                jnp.sum((r2 >= t).astype(jnp.int32)))

    def _cnt_full(i, t):
        return jnp.sum((yis[i] >= t).astype(jnp.int32))

    def _step_red(i, carry, probe):
        lo, hi = carry
        take = _cnt_red(i, probe) >= _K
        return (jnp.where(take, probe, lo), jnp.where(take, hi, probe))

    def _bis(_, carry):
        return tuple(_step_red(i, carry[i], (carry[i][0] + carry[i][1]) // 2)
                     for i in range(_IMGS))

    init0 = (jnp.int32(0), jnp.int32(_ONE_BITS))
    seeded = tuple(_step_red(i, init0, jnp.int32(_HALF_BITS))
                   for i in range(_IMGS))
    cand = lax.fori_loop(0, 23, _bis, seeded)

    def _full_bisect(i):
        def _stepf(_, carry):
            lo, hi, clo, chi = carry
            mid = (lo + hi) // 2
            c = _cnt_full(i, mid)
            take = c >= _K
            return (jnp.where(take, mid, lo), jnp.where(take, hi, mid),
                    jnp.where(take, c, clo), jnp.where(take, chi, c))

        lo, _, clo, chi = lax.fori_loop(
            0, 31, _stepf,
            (jnp.int32(0), jnp.int32(_ONE_BITS), jnp.int32(h * w),
             jnp.int32(0)))
        return lo, clo, chi

    res = []
    for i in range(_IMGS):
        lo = cand[i][0]
        c1 = _cnt_full(i, lo)
        c2 = _cnt_full(i, lo + 1)
        res.append(lax.cond((c1 >= _K) & (c2 < _K),
                            lambda lo=lo, c1=c1, c2=c2: (lo, c1, c2),
                            lambda i=i: _full_bisect(i)))

    for i in range(_IMGS):
        t_star, cnt_ge_star, cnt_gt = res[i]
        tmask = _topk_mask(yis[i], cnt_ge_star, cnt_gt, t_star, h, w)
        tmask_ref[i] = tmask.astype(jnp.int8)
        y = lax.bitcast_convert_type(yis[i], jnp.float32)
        out_ref[i] = _gauss5(jnp.where(tmask, y, 0.0), h, w)


def kernel(im1w_score):
    b, h, w, c = im1w_score.shape
    x = im1w_score.reshape(b, h, w)
    spec = pl.BlockSpec((_IMGS, h, w), lambda i: (i, 0, 0))
    out, tmask, topkv = pl.pallas_call(
        _body,
        grid=(b // _IMGS,),
        in_specs=[spec],
        out_specs=[spec, spec, spec],
        out_shape=[
            jax.ShapeDtypeStruct((b, h, w), jnp.float32),
            jax.ShapeDtypeStruct((b, h, w), jnp.int8),
            jax.ShapeDtypeStruct((b, h, w), jnp.float32),
        ],
        compiler_params=pltpu.CompilerParams(
            dimension_semantics=("arbitrary",)),
    )(x)
    return (out.reshape(b, h, w, c),
            tmask.reshape(b, h, w, c).astype(jnp.bool_),
            topkv.reshape(b, h, w, c))


# speedup vs baseline: 1.3677x; 1.0523x over previous
import numpy as np
import jax
import jax.numpy as jnp
from jax import lax
from jax.experimental import pallas as pl
from jax.experimental.pallas import tpu as pltpu

_K = 512
_BORDER = 8
_R_G = 2
_GAUSS = np.exp(-2.0 * (np.arange(-_R_G, _R_G + 1) ** 2)).astype(np.float32)
_ONE_BITS = 0x3F800000
_HALF_BITS = 0x3F000000
_IMGS = 4


def _nms_survivors(x, h, w):
    row = lax.broadcasted_iota(jnp.int32, (h, w), 0)
    col = lax.broadcasted_iota(jnp.int32, (h, w), 1)
    span = jnp.uint32(h - 2 * _BORDER)
    inb = ((row - _BORDER).astype(jnp.uint32) < span) & \
          ((col - _BORDER).astype(jnp.uint32) < span)
    xt = jnp.where(inb, x, 0.0)

    p1 = jnp.maximum(xt, pltpu.roll(xt, w - 1, 1))
    p3 = jnp.maximum(p1, pltpu.roll(p1, w - 2, 1))
    m1 = jnp.maximum(pltpu.roll(p3, 2, 1),
                     pltpu.roll(xt, w - 2, 1))
    q1 = jnp.maximum(m1, pltpu.roll(m1, h - 1, 0))
    q3 = jnp.maximum(q1, pltpu.roll(q1, h - 2, 0))
    mx = jnp.maximum(pltpu.roll(q3, 2, 0),
                     pltpu.roll(m1, h - 2, 0))
    return jnp.where(xt >= mx, xt, 0.0)


def _topk_mask(yi, cnt_ge_star, cnt_gt, t_star, h, w):
    ties = yi == t_star
    row = lax.broadcasted_iota(jnp.int32, (h, w), 0)
    col = lax.broadcasted_iota(jnp.int32, (h, w), 1)
    flat = row * w + col

    def _tie_search():
        need = _K - cnt_gt

        def _bis2(_, lo_hi):
            lo, hi = lo_hi
            mid = (lo + hi) // 2
            take = jnp.sum((ties & (flat <= mid)).astype(jnp.int32)) >= need
            return jnp.where(take, lo, mid + 1), jnp.where(take, mid, hi)

        return lax.fori_loop(0, 18, _bis2,
                             (jnp.int32(0), jnp.int32(h * w - 1)))[1]

    m_star = lax.cond(cnt_ge_star == _K,
                      lambda: jnp.int32(h * w - 1), _tie_search)
    return (yi > t_star) | (ties & (flat <= m_star))


def _gauss5(z, h, w):
    t1 = z * _GAUSS[_R_G]
    for d in range(1, _R_G + 1):
        t1 = t1 + _GAUSS[_R_G + d] * (pltpu.roll(z, d, 1) +
                                      pltpu.roll(z, w - d, 1))
    o = t1 * _GAUSS[_R_G]
    for d in range(1, _R_G + 1):
        o = o + _GAUSS[_R_G + d] * (pltpu.roll(t1, d, 0) +
                                    pltpu.roll(t1, h - d, 0))
    return jnp.clip(o, 0.0, 1.0)


def _body(x_ref, out_ref, tmask_ref, topkv_ref):
    h, w = x_ref.shape[1], x_ref.shape[2]

    yis, red = [], []
    for i in range(_IMGS):
        y = _nms_survivors(x_ref[i], h, w)
        topkv_ref[i] = y
        yi = lax.bitcast_convert_type(y, jnp.int32)
        yis.append(yi)

        def _top2_merge(a, b):
            a1, a2 = a
            b1, b2 = b
            t1 = jnp.maximum(a1, b1)
            t2 = jnp.maximum(jnp.minimum(a1, b1),
                             jnp.where(a1 >= b1, a2, b2))
            return t1, t2

        q = h // 8
        slabs = [yi[k * q:(k + 1) * q] for k in range(8)]
        pairs = [(jnp.maximum(slabs[k], slabs[k + 1]),
                  jnp.minimum(slabs[k], slabs[k + 1]))
                 for k in range(0, 8, 2)]
        r1, r2 = _top2_merge(_top2_merge(pairs[0], pairs[1]),
                             _top2_merge(pairs[2], pairs[3]))
        red.append((r1, r2))

    def _cnt_red(i, t):
        r1, r2 = red[i]
        return (jnp.sum((r1 >= t).astype(jnp.int32)) +
                jnp.sum((r2 >= t).astype(jnp.int32)))

    def _cnt_full(i, t):
        return jnp.sum((yis[i] >= t).astype(jnp.int32))

    def _step_red(i, carry, probe):
        lo, hi = carry
        take = _cnt_red(i, probe) >= _K
        return (jnp.where(take, probe, lo), jnp.where(take, hi, probe))

    def _bis(_, carry):
        return tuple(_step_red(i, carry[i], (carry[i][0] + carry[i][1]) // 2)
                     for i in range(_IMGS))

    init0 = (jnp.int32(0), jnp.int32(_ONE_BITS))
    seeded = tuple(_step_red(i, init0, jnp.int32(_HALF_BITS))
                   for i in range(_IMGS))
    cand = lax.fori_loop(0, 23, _bis, seeded)

    def _full_bisect(i):
        def _stepf(_, carry):
            lo, hi, clo, chi = carry
            mid = (lo + hi) // 2
            c = _cnt_full(i, mid)
            take = c >= _K
            return (jnp.where(take, mid, lo), jnp.where(take, hi, mid),
                    jnp.where(take, c, clo), jnp.where(take, chi, c))

        lo, _, clo, chi = lax.fori_loop(
            0, 31, _stepf,
            (jnp.int32(0), jnp.int32(_ONE_BITS), jnp.int32(h * w),
             jnp.int32(0)))
        return lo, clo, chi

    res = []
    for i in range(_IMGS):
        lo = cand[i][0]
        c1 = _cnt_full(i, lo)
        c2 = _cnt_full(i, lo + 1)
        res.append(lax.cond((c1 >= _K) & (c2 < _K),
                            lambda lo=lo, c1=c1, c2=c2: (lo, c1, c2),
                            lambda i=i: _full_bisect(i)))

    for i in range(_IMGS):
        t_star, cnt_ge_star, cnt_gt = res[i]
        tmask = _topk_mask(yis[i], cnt_ge_star, cnt_gt, t_star, h, w)
        tmask_ref[i] = tmask.astype(jnp.int8)
        y = lax.bitcast_convert_type(yis[i], jnp.float32)
        out_ref[i] = _gauss5(jnp.where(tmask, y, 0.0), h, w)


def kernel(im1w_score):
    b, h, w, c = im1w_score.shape
    x = im1w_score.reshape(b, h, w)
    spec = pl.BlockSpec((_IMGS, h, w), lambda i: (i, 0, 0))
    out, tmask, topkv = pl.pallas_call(
        _body,
        grid=(b // _IMGS,),
        in_specs=[spec],
        out_specs=[spec, spec, spec],
        out_shape=[
            jax.ShapeDtypeStruct((b, h, w), jnp.float32),
            jax.ShapeDtypeStruct((b, h, w), jnp.int8),
            jax.ShapeDtypeStruct((b, h, w), jnp.float32),
        ],
        compiler_params=pltpu.CompilerParams(
            dimension_semantics=("arbitrary",)),
    )(x)
    return (out.reshape(b, h, w, c),
            tmask.reshape(b, h, w, c).astype(jnp.bool_),
            topkv.reshape(b, h, w, c))
